# baseline (speedup 1.0000x reference)
import functools

import jax
import jax.numpy as jnp
from jax import lax
from jax.experimental import pallas as pl
from jax.experimental.pallas import tpu as pltpu
from jax.experimental.pallas import tpu_sc as plsc

_NUM_WORKERS = 32
_NSUB = 16
_CHUNK = 8
_TW = 16
_BW = 32
_LANES = 16


def _sc_prompt_fill(table, labels, tmpl5, crows):
    b = labels.shape[0]
    _, n_gen, d = table.shape
    n_sh = tmpl5.shape[0]
    n_own_tot = crows.shape[0]
    seq = 1 + n_gen + (n_sh - 1) + n_own_tot
    b_per_w = b // _NUM_WORKERS
    n_chunks = b_per_w // _CHUNK
    lanes_per_row = d // _LANES
    own_t0 = seq - n_own_tot

    mesh = plsc.VectorSubcoreMesh(core_axis_name="c", subcore_axis_name="s")

    @functools.partial(
        pl.kernel,
        mesh=mesh,
        out_type=jax.ShapeDtypeStruct((seq, b, d), jnp.float32),
        scratch_types=[
            pltpu.VMEM((b_per_w,), jnp.int32),
            pltpu.VMEM((2, 1, d), jnp.float32),
            pltpu.VMEM((1, _BW, d), jnp.float32),
            pltpu.VMEM((1, _BW, d), jnp.float32),
            pltpu.VMEM((_CHUNK, n_gen, d), jnp.float32),
            pltpu.VMEM((n_gen, _CHUNK, d), jnp.float32),
            pltpu.VMEM_SHARED((n_sh, _TW, d), jnp.float32),
            pltpu.SemaphoreType.DMA,
            pltpu.SemaphoreType.DMA,
            pltpu.SemaphoreType.DMA,
            pltpu.SemaphoreType.DMA,
        ],
    )
    def k(table_hbm, label_hbm, tmpl5_hbm, crows_hbm, out_hbm,
          idx_v, crows_v, bc0_v, bc1_v, stage_v, genbuf_v, shared,
          gsem, wsem, tsem, vsem):
        cid = lax.axis_index("c")
        sid = lax.axis_index("s")
        wid = sid * 2 + cid
        base = pl.multiple_of(wid * b_per_w, b_per_w)
        pltpu.sync_copy(label_hbm.at[pl.ds(base, b_per_w)], idx_v)
        pltpu.sync_copy(crows_hbm.at[pl.ds(wid * 2, 2)], crows_v)

        @pl.when(sid < n_sh)
        def _load_shared():
            pltpu.sync_copy(tmpl5_hbm.at[pl.ds(sid, 1)],
                            shared.at[pl.ds(sid, 1)])

        for i, bc in enumerate((bc0_v, bc1_v)):
            vals = [crows_v[i, 0, pl.ds(l * _LANES, _LANES)]
                    for l in range(lanes_per_row)]
            for r in range(_BW):
                for l in range(lanes_per_row):
                    bc[0, r, pl.ds(l * _LANES, _LANES)] = vals[l]

        own_waits = []
        for i, bc in enumerate((bc0_v, bc1_v)):
            t_own = own_t0 + wid * 2 + i
            for h in range(b // _BW):
                dsc = pltpu.make_async_copy(
                    bc, out_hbm.at[pl.ds(t_own, 1), pl.ds(h * _BW, _BW), :],
                    wsem)
                dsc.start()
                own_waits.append(dsc)

        plsc.subcore_barrier()

        for h in range(b_per_w // _TW):
            dst_b = pl.ds(base + h * _TW, _TW)
            pltpu.make_async_copy(
                shared.at[pl.ds(0, 1)],
                out_hbm.at[pl.ds(0, 1), dst_b, :], tsem).start()
            pltpu.make_async_copy(
                shared.at[pl.ds(1, n_sh - 1)],
                out_hbm.at[pl.ds(1 + n_gen, n_sh - 1), dst_b, :],
                tsem).start()

        def chunk_body(c, carry):
            coff = pl.multiple_of(c * _CHUNK, _CHUNK)
            pltpu.async_copy(
                table_hbm.at[idx_v.at[pl.ds(coff, _CHUNK)]],
                stage_v, gsem).wait()

            @pl.when(c > 0)
            def _drain_prev():
                pltpu.make_async_copy(
                    table_hbm.at[pl.ds(0, _CHUNK)], genbuf_v, vsem).wait()

            def trans_body(r, carry2):
                for j in range(n_gen):
                    for l in range(lanes_per_row):
                        genbuf_v[j, r, pl.ds(l * _LANES, _LANES)] = (
                            stage_v[r, j, pl.ds(l * _LANES, _LANES)])
                return carry2

            lax.fori_loop(0, _CHUNK, trans_body, 0)
            pltpu.make_async_copy(
                genbuf_v,
                out_hbm.at[pl.ds(1, n_gen), pl.ds(base + coff, _CHUNK), :],
                vsem).start()
            return carry

        lax.fori_loop(0, n_chunks, chunk_body, 0)

        pltpu.make_async_copy(
            table_hbm.at[pl.ds(0, _CHUNK)], genbuf_v, vsem).wait()
        for dsc in own_waits:
            dsc.wait()
        for _ in range(b_per_w // _TW):
            pltpu.make_async_copy(tmpl5_hbm, shared, tsem).wait()

    return k(table, labels, tmpl5, crows)


def kernel(label, ctx_generic, ctx_modality, ctx_platform,
           token_prefix, token_suffix):
    n_gen = ctx_generic.shape[1]
    d = ctx_generic.shape[2]
    n_zero = ctx_modality.shape[1] + ctx_platform.shape[1]
    n_suf = token_suffix.shape[1]
    n_own = 2 * _NUM_WORKERS
    n_zero_own = n_own - n_suf
    n_zero_sh = n_zero - n_zero_own

    tmpl5 = jnp.concatenate([
        jnp.broadcast_to(token_prefix.astype(jnp.float32),
                         (token_prefix.shape[1], _TW, d)),
        jnp.zeros((n_zero_sh, _TW, d), jnp.float32),
    ], axis=0)
    crows = jnp.concatenate([
        jnp.zeros((n_zero_own, 1, d), jnp.float32),
        jnp.transpose(token_suffix.astype(jnp.float32), (1, 0, 2)),
    ], axis=0)
    slabbed = _sc_prompt_fill(ctx_generic, label.astype(jnp.int32),
                              tmpl5, crows)
    return jnp.transpose(slabbed, (1, 0, 2))


# speedup vs baseline: 1.1581x; 1.1581x over previous
import functools

import jax
import jax.numpy as jnp
from jax import lax
from jax.experimental import pallas as pl
from jax.experimental.pallas import tpu as pltpu
from jax.experimental.pallas import tpu_sc as plsc

_NUM_WORKERS = 32
_NSUB = 16
_CHUNK = 8
_TW = 16
_BW = 32
_LANES = 16


def _sc_prompt_fill(table2, labels, tmpl5, crows, n_gen):
    b = labels.shape[0]
    d = table2.shape[1]
    n_sh = tmpl5.shape[0]
    n_own_tot = crows.shape[0]
    seq = 1 + n_gen + (n_sh - 1) + n_own_tot
    b_per_w = b // _NUM_WORKERS
    n_chunks = b_per_w // _CHUNK
    lanes_per_row = d // _LANES
    own_t0 = seq - n_own_tot

    mesh = plsc.VectorSubcoreMesh(core_axis_name="c", subcore_axis_name="s")

    @functools.partial(
        pl.kernel,
        mesh=mesh,
        out_type=jax.ShapeDtypeStruct((seq, b, d), jnp.float32),
        scratch_types=(
            [pltpu.VMEM((b_per_w,), jnp.int32)]
            + [pltpu.VMEM((b_per_w,), jnp.int32) for _ in range(n_gen)]
            + [pltpu.VMEM((2, 1, d), jnp.float32),
               pltpu.VMEM((1, _BW, d), jnp.float32),
               pltpu.VMEM((1, _BW, d), jnp.float32)]
            + [pltpu.VMEM((_CHUNK, d), jnp.float32) for _ in range(n_gen)]
            + [pltpu.VMEM_SHARED((n_sh, _TW, d), jnp.float32),
               pltpu.SemaphoreType.DMA,
               pltpu.SemaphoreType.DMA,
               pltpu.SemaphoreType.DMA,
               pltpu.SemaphoreType.DMA]
        ),
    )
    def k(table_hbm, label_hbm, tmpl5_hbm, crows_hbm, out_hbm,
          idx_v, *rest):
        idx8s = rest[:n_gen]
        crows_v, bc0_v, bc1_v = rest[n_gen:n_gen + 3]
        gbufs = rest[n_gen + 3:2 * n_gen + 3]
        shared, gsem, wsem, tsem, vsem = rest[2 * n_gen + 3:]
        cid = lax.axis_index("c")
        sid = lax.axis_index("s")
        wid = sid * 2 + cid
        base = pl.multiple_of(wid * b_per_w, b_per_w)
        pltpu.sync_copy(label_hbm.at[pl.ds(base, b_per_w)], idx_v)
        pltpu.sync_copy(crows_hbm.at[pl.ds(wid * 2, 2)], crows_v)

        @pl.when(sid < n_sh)
        def _load_shared():
            pltpu.sync_copy(tmpl5_hbm.at[pl.ds(sid, 1)],
                            shared.at[pl.ds(sid, 1)])

        for j in range(n_gen):
            for half in range(b_per_w // _LANES):
                sl = pl.ds(half * _LANES, _LANES)
                idx8s[j][sl] = idx_v[sl] * n_gen + j

        for i, bc in enumerate((bc0_v, bc1_v)):
            vals = [crows_v[i, 0, pl.ds(l * _LANES, _LANES)]
                    for l in range(lanes_per_row)]
            for r in range(_BW):
                for l in range(lanes_per_row):
                    bc[0, r, pl.ds(l * _LANES, _LANES)] = vals[l]

        own_waits = []
        for i, bc in enumerate((bc0_v, bc1_v)):
            t_own = own_t0 + wid * 2 + i
            for h in range(b // _BW):
                dsc = pltpu.make_async_copy(
                    bc, out_hbm.at[pl.ds(t_own, 1), pl.ds(h * _BW, _BW), :],
                    wsem)
                dsc.start()
                own_waits.append(dsc)

        plsc.subcore_barrier()

        for h in range(b_per_w // _TW):
            dst_b = pl.ds(base + h * _TW, _TW)
            pltpu.make_async_copy(
                shared.at[pl.ds(0, 1)],
                out_hbm.at[pl.ds(0, 1), dst_b, :], tsem).start()
            pltpu.make_async_copy(
                shared.at[pl.ds(1, n_sh - 1)],
                out_hbm.at[pl.ds(1 + n_gen, n_sh - 1), dst_b, :],
                tsem).start()

        def chunk_body(c, carry):
            coff = pl.multiple_of(c * _CHUNK, _CHUNK)

            @pl.when(c > 0)
            def _drain_prev():
                for j in range(n_gen):
                    pltpu.make_async_copy(
                        table_hbm.at[pl.ds(0, _CHUNK)], gbufs[j],
                        vsem).wait()

            gds = []
            for j in range(n_gen):
                gd = pltpu.make_async_copy(
                    table_hbm.at[idx8s[j].at[pl.ds(coff, _CHUNK)]],
                    gbufs[j], gsem)
                gd.start()
                gds.append(gd)
            for j in range(n_gen):
                gds[j].wait()
                pltpu.make_async_copy(
                    gbufs[j],
                    out_hbm.at[1 + j, pl.ds(base + coff, _CHUNK), :],
                    vsem).start()
            return carry

        lax.fori_loop(0, n_chunks, chunk_body, 0)

        for j in range(n_gen):
            pltpu.make_async_copy(
                table_hbm.at[pl.ds(0, _CHUNK)], gbufs[j], vsem).wait()
        for dsc in own_waits:
            dsc.wait()
        for _ in range(b_per_w // _TW):
            pltpu.make_async_copy(tmpl5_hbm, shared, tsem).wait()

    return k(table2, labels, tmpl5, crows)


def kernel(label, ctx_generic, ctx_modality, ctx_platform,
           token_prefix, token_suffix):
    num_class, n_gen, d = ctx_generic.shape
    n_zero = ctx_modality.shape[1] + ctx_platform.shape[1]
    n_suf = token_suffix.shape[1]
    n_own = 2 * _NUM_WORKERS
    n_zero_own = n_own - n_suf
    n_zero_sh = n_zero - n_zero_own

    table2 = ctx_generic.reshape(num_class * n_gen, d)
    tmpl5 = jnp.concatenate([
        jnp.broadcast_to(token_prefix.astype(jnp.float32),
                         (token_prefix.shape[1], _TW, d)),
        jnp.zeros((n_zero_sh, _TW, d), jnp.float32),
    ], axis=0)
    crows = jnp.concatenate([
        jnp.zeros((n_zero_own, 1, d), jnp.float32),
        jnp.transpose(token_suffix.astype(jnp.float32), (1, 0, 2)),
    ], axis=0)
    slabbed = _sc_prompt_fill(table2, label.astype(jnp.int32),
                              tmpl5, crows, n_gen)
    return jnp.transpose(slabbed, (1, 0, 2))
